# baseline (speedup 1.0000x reference)
import jax
import jax.numpy as jnp
from jax.experimental import pallas as pl
from jax.experimental.pallas import tpu as pltpu

EPS = 1e-05


def _block_stats(ref):
    C = ref.shape[1]
    v = ref[0, 0:8]
    s = v
    q = v * v
    for k in range(1, C // 8):
        v = ref[0, 8 * k : 8 * k + 8]
        s = s + v
        q = q + v * v
    return jnp.sum(s, axis=0), jnp.sum(q, axis=0)


def _spe_kernel(s_ref, xs_ref, xn_ref, out_ref):
    C = xn_ref.shape[1]

    sum_n, sumsq_n = _block_stats(xn_ref)
    sum_s, sumsq_s = _block_stats(xs_ref)

    mean_n = sum_n * (1.0 / C)
    var_n = (sumsq_n - sum_n * mean_n) * (1.0 / (C - 1))
    mean_s = sum_s * (1.0 / C)
    var_s = (sumsq_s - sum_s * mean_s) * (1.0 / (C - 1))

    ratio = jnp.sqrt((var_n + EPS) / (var_s + EPS))
    offset = mean_n - mean_s * ratio

    for k in range(C // 8):
        sl = slice(8 * k, 8 * k + 8)
        out_ref[0, sl] = xs_ref[0, sl] * ratio + offset


def kernel(x, idx_swap):
    N, C, H, W = x.shape

    grid_spec = pltpu.PrefetchScalarGridSpec(
        num_scalar_prefetch=1,
        grid=(N,),
        in_specs=[
            pl.BlockSpec((1, C, H, W), lambda n, s: (s[n], 0, 0, 0)),
            pl.BlockSpec((1, C, H, W), lambda n, s: (n, 0, 0, 0)),
        ],
        out_specs=pl.BlockSpec((1, C, H, W), lambda n, s: (n, 0, 0, 0)),
    )

    return pl.pallas_call(
        _spe_kernel,
        grid_spec=grid_spec,
        out_shape=jax.ShapeDtypeStruct((N, C, H, W), jnp.float32),
    )(idx_swap, x, x)


# speedup vs baseline: 1.8744x; 1.8744x over previous
import jax
import jax.numpy as jnp
from jax.experimental import pallas as pl
from jax.experimental.pallas import tpu as pltpu

EPS = 1e-05


def _block_stats(ref):
    C = ref.shape[1]
    v = ref[0, 0:8]
    s = v
    q = v * v
    for k in range(1, C // 8):
        v = ref[0, 8 * k : 8 * k + 8]
        s = s + v
        q = q + v * v
    return jnp.sum(s, axis=0), jnp.sum(q, axis=0)


def _spe_kernel(s_ref, xs_ref, xn_ref, out_ref):
    C = xn_ref.shape[1]

    sum_n, sumsq_n = _block_stats(xn_ref)
    sum_s, sumsq_s = _block_stats(xs_ref)

    mean_n = sum_n * (1.0 / C)
    var_n = (sumsq_n - sum_n * mean_n) * (1.0 / (C - 1))
    mean_s = sum_s * (1.0 / C)
    var_s = (sumsq_s - sum_s * mean_s) * (1.0 / (C - 1))

    ratio = jnp.sqrt((var_n + EPS) / (var_s + EPS))
    offset = mean_n - mean_s * ratio

    for k in range(C // 8):
        sl = slice(8 * k, 8 * k + 8)
        out_ref[0, sl] = xs_ref[0, sl] * ratio + offset


def kernel(x, idx_swap):
    N, C, H, W = x.shape
    HW = H * W
    xv = x.reshape(N, C, HW // 128, 128)

    grid_spec = pltpu.PrefetchScalarGridSpec(
        num_scalar_prefetch=1,
        grid=(N,),
        in_specs=[
            pl.BlockSpec((1, C, HW // 128, 128), lambda n, s: (s[n], 0, 0, 0)),
            pl.BlockSpec((1, C, HW // 128, 128), lambda n, s: (n, 0, 0, 0)),
        ],
        out_specs=pl.BlockSpec((1, C, HW // 128, 128), lambda n, s: (n, 0, 0, 0)),
    )

    out = pl.pallas_call(
        _spe_kernel,
        grid_spec=grid_spec,
        out_shape=jax.ShapeDtypeStruct((N, C, HW // 128, 128), jnp.float32),
    )(idx_swap, xv, xv)
    return out.reshape(N, C, H, W)
